# baseline (speedup 1.0000x reference)
import jax
import jax.numpy as jnp
import numpy as np
from jax import lax
from jax.experimental import pallas as pl
from jax.experimental.pallas import tpu as pltpu

B = 16
C = 32
N = 32
J = 64
CHUNK = 512
K = J * CHUNK

_GUMBEL_CBN = np.asarray(
    jax.random.gumbel(jax.random.key(123), (B, C, N), jnp.float32)
).transpose(1, 0, 2).copy()


def _sum_body(filt_ref, x_ref, s_ref, acc):
    j = pl.program_id(0)

    @pl.when(j == 0)
    def _():
        acc[...] = jnp.zeros_like(acc)

    xb = x_ref[:, 0, 0, :]
    acc[...] += jnp.sum(xb, axis=1, keepdims=True)

    @pl.when(j == J - 1)
    def _():
        s_ref[...] = acc[...]


def _main_body(filt_ref, s_ref, x_ref, w_ref, g_ref, o_ref, h3, inv_s):
    j = pl.program_id(0)

    @pl.when(j == 0)
    def _():
        h3[...] = jnp.zeros_like(h3)
        inv_s[...] = s_ref[...]

    xb = x_ref[:, 0, 0, :]
    s = inv_s[...]
    xn = jnp.where(s > 0.0, xb / s, 0.0)
    for c in range(C):
        wc = w_ref[c, 0, :, :]
        h3[c, :, :] += jnp.dot(xn, wc, preferred_element_type=jnp.float32)

    @pl.when(j == J - 1)
    def _():
        total = jnp.float32(0.0)
        for c in range(C):
            total += jnp.sum(jnp.max(h3[c, :, :], axis=1))
        avg = total / jnp.float32(B * C)
        temp = 1.0 / (avg + jnp.float32(0.0001)) - 1.0
        iota2 = lax.broadcasted_iota(jnp.int32, (B, N), 1)
        for c in range(C):
            z = h3[c, :, :] / temp + g_ref[c, :, :]
            m = jnp.max(z, axis=1, keepdims=True)
            cand = jnp.where(z == m, iota2, N)
            am = jnp.min(cand, axis=1, keepdims=True)
            o_ref[:, c, :] = (iota2 == am).astype(jnp.int32)


def kernel(x, weights, input_filter):
    x4 = x.reshape(B, 1024, 1, CHUNK)
    w4 = weights.reshape(C, J, CHUNK, N)
    g3 = jnp.asarray(_GUMBEL_CBN)

    row_sums = pl.pallas_call(
        _sum_body,
        grid_spec=pltpu.PrefetchScalarGridSpec(
            num_scalar_prefetch=1,
            grid=(J,),
            in_specs=[
                pl.BlockSpec((B, 1, 1, CHUNK),
                             lambda j, filt: (0, filt[j], 0, 0)),
            ],
            out_specs=pl.BlockSpec((B, 1), lambda j, filt: (0, 0)),
            scratch_shapes=[pltpu.VMEM((B, 1), jnp.float32)],
        ),
        out_shape=jax.ShapeDtypeStruct((B, 1), jnp.float32),
    )(input_filter, x4)

    out = pl.pallas_call(
        _main_body,
        grid_spec=pltpu.PrefetchScalarGridSpec(
            num_scalar_prefetch=1,
            grid=(J,),
            in_specs=[
                pl.BlockSpec((B, 1), lambda j, filt: (0, 0)),
                pl.BlockSpec((B, 1, 1, CHUNK),
                             lambda j, filt: (0, filt[j], 0, 0)),
                pl.BlockSpec((C, 1, CHUNK, N), lambda j, filt: (0, j, 0, 0)),
                pl.BlockSpec((C, B, N), lambda j, filt: (0, 0, 0)),
            ],
            out_specs=pl.BlockSpec((B, C, N), lambda j, filt: (0, 0, 0)),
            scratch_shapes=[
                pltpu.VMEM((C, B, N), jnp.float32),
                pltpu.VMEM((B, 1), jnp.float32),
            ],
        ),
        out_shape=jax.ShapeDtypeStruct((B, C, N), jnp.int32),
    )(input_filter, row_sums, x4, w4, g3)

    return out
